# baseline (speedup 1.0000x reference)
import functools
import jax
import jax.numpy as jnp
from jax import lax
from jax.experimental import pallas as pl
from jax.experimental.pallas import tpu as pltpu
from jax.experimental.pallas import tpu_sc as plsc

D_MODEL = 64
SQRT_D = 8.0

NC = 2
NS = 16
NW = NC * NS

CHUNK = 512


def _make_kernel(B):
    assert B % (8 * NW) == 0
    bpw = B // NW
    assert bpw % CHUNK == 0
    nchunk = bpw // CHUNK
    mesh = plsc.VectorSubcoreMesh(core_axis_name="c", subcore_axis_name="s")

    @functools.partial(
        pl.kernel,
        mesh=mesh,
        out_type=jax.ShapeDtypeStruct((B, D_MODEL), jnp.float32),
        scratch_types=[
            pltpu.VMEM((bpw,), jnp.int32),
            pltpu.VMEM((CHUNK, D_MODEL), jnp.float32),
            pltpu.SemaphoreType.DMA,
        ],
        compiler_params=pltpu.CompilerParams(use_tc_tiling_on_sc=False),
    )
    def k(x_hbm, table_hbm, out_hbm, idx_v, rows_v, gsem):
        wid = lax.axis_index("s") * NC + lax.axis_index("c")
        base = wid * bpw
        pltpu.sync_copy(x_hbm.at[pl.ds(base, bpw)], idx_v)

        def chunk_body(g, _):
            off = g * CHUNK
            pltpu.async_copy(
                table_hbm.at[idx_v.at[pl.ds(off, CHUNK)]], rows_v, gsem
            ).wait()

            def scale_body(i, _):
                for j in range(D_MODEL // 16):
                    sl = pl.ds(j * 16, 16)
                    rows_v[i, sl] = rows_v[i, sl] * SQRT_D
                return _

            lax.fori_loop(0, CHUNK, scale_body, None)
            pltpu.sync_copy(rows_v, out_hbm.at[pl.ds(base + off, CHUNK)])
            return _

        lax.fori_loop(0, nchunk, chunk_body, None)

    return k


def kernel(x, table):
    B = x.shape[0] * x.shape[1]
    idx = x.reshape(B).astype(jnp.int32)
    out = _make_kernel(B)(idx, table)
    return out.reshape(x.shape[0], x.shape[1], D_MODEL)


# speedup vs baseline: 1.0862x; 1.0862x over previous
import functools
import jax
import jax.numpy as jnp
from jax import lax
from jax.experimental import pallas as pl
from jax.experimental.pallas import tpu as pltpu
from jax.experimental.pallas import tpu_sc as plsc

D_MODEL = 64
SQRT_D = 8.0

NC = 2
NS = 16
NW = NC * NS

CHUNK = 512


def _make_kernel(B):
    assert B % (8 * NW) == 0
    bpw = B // NW
    assert bpw % (2 * CHUNK) == 0
    npair = bpw // (2 * CHUNK)
    nchunk = 2 * npair
    mesh = plsc.VectorSubcoreMesh(core_axis_name="c", subcore_axis_name="s")

    @functools.partial(
        pl.kernel,
        mesh=mesh,
        out_type=jax.ShapeDtypeStruct((B, D_MODEL), jnp.float32),
        scratch_types=[
            pltpu.VMEM((bpw,), jnp.int32),
            pltpu.VMEM((CHUNK, D_MODEL), jnp.float32),
            pltpu.VMEM((CHUNK, D_MODEL), jnp.float32),
            pltpu.SemaphoreType.DMA,
            pltpu.SemaphoreType.DMA,
            pltpu.SemaphoreType.DMA,
            pltpu.SemaphoreType.DMA,
        ],
        compiler_params=pltpu.CompilerParams(use_tc_tiling_on_sc=False),
    )
    def k(x_hbm, table_hbm, out_hbm, idx_v, rows0, rows1, g0, g1, o0, o1):
        wid = lax.axis_index("s") * NC + lax.axis_index("c")
        base = wid * bpw
        pltpu.sync_copy(x_hbm.at[pl.ds(base, bpw)], idx_v)
        rows = (rows0, rows1)
        gsem = (g0, g1)
        osem = (o0, o1)

        def _gather_refs(g, b):
            return (
                table_hbm.at[idx_v.at[pl.ds(g * CHUNK, CHUNK)]],
                rows[b],
            )

        def _wback_refs(g, b):
            return (
                rows[b],
                out_hbm.at[pl.ds(base + g * CHUNK, CHUNK)],
            )

        def gather_start(g, b):
            src, dst = _gather_refs(g, b)
            pltpu.async_copy(src, dst, gsem[b])

        def gather_wait(g, b):
            src, dst = _gather_refs(g, b)
            pltpu.make_async_copy(src, dst, gsem[b]).wait()

        def wback_start(g, b):
            src, dst = _wback_refs(g, b)
            pltpu.async_copy(src, dst, osem[b])

        def wback_wait(g, b):
            src, dst = _wback_refs(g, b)
            pltpu.make_async_copy(src, dst, osem[b]).wait()

        def scale(b):
            def scale_body(i, _):
                for j in range(D_MODEL // 16):
                    sl = pl.ds(j * 16, 16)
                    rows[b][i, sl] = rows[b][i, sl] * SQRT_D
                return _

            lax.fori_loop(0, CHUNK, scale_body, None)

        gather_start(0, 0)

        def pair_body(p, _):
            for b in range(2):
                g = 2 * p + b
                if b == 0:

                    @pl.when(p > 0)
                    def _():
                        wback_wait(g - 1, 1)

                    gather_start(g + 1, 1)
                else:

                    @pl.when(p < npair - 1)
                    def _():
                        wback_wait(g - 1, 0)
                        gather_start(g + 1, 0)

                gather_wait(g, b)
                scale(b)
                wback_start(g, b)
            return _

        lax.fori_loop(0, npair, pair_body, None)
        wback_wait(nchunk - 2, 0)
        wback_wait(nchunk - 1, 1)

    return k


def kernel(x, table):
    B = x.shape[0] * x.shape[1]
    idx = x.reshape(B).astype(jnp.int32)
    out = _make_kernel(B)(idx, table)
    return out.reshape(x.shape[0], x.shape[1], D_MODEL)


# speedup vs baseline: 1.0886x; 1.0023x over previous
import functools
import jax
import jax.numpy as jnp
from jax import lax
from jax.experimental import pallas as pl
from jax.experimental.pallas import tpu as pltpu
from jax.experimental.pallas import tpu_sc as plsc

D_MODEL = 64
SQRT_D = 8.0

NC = 2
NS = 16
NW = NC * NS

CHUNK = 640


def _make_kernel(B):
    assert B % (8 * NW) == 0
    bpw = B // NW
    assert bpw % (2 * CHUNK) == 0
    npair = bpw // (2 * CHUNK)
    nchunk = 2 * npair
    mesh = plsc.VectorSubcoreMesh(core_axis_name="c", subcore_axis_name="s")

    @functools.partial(
        pl.kernel,
        mesh=mesh,
        out_type=jax.ShapeDtypeStruct((B, D_MODEL), jnp.float32),
        scratch_types=[
            pltpu.VMEM((bpw,), jnp.int32),
            pltpu.VMEM((CHUNK, D_MODEL), jnp.float32),
            pltpu.VMEM((CHUNK, D_MODEL), jnp.float32),
            pltpu.SemaphoreType.DMA,
            pltpu.SemaphoreType.DMA,
            pltpu.SemaphoreType.DMA,
            pltpu.SemaphoreType.DMA,
        ],
        compiler_params=pltpu.CompilerParams(use_tc_tiling_on_sc=False),
    )
    def k(x_hbm, table_hbm, out_hbm, idx_v, rows0, rows1, g0, g1, o0, o1):
        wid = lax.axis_index("s") * NC + lax.axis_index("c")
        base = wid * bpw
        pltpu.sync_copy(x_hbm.at[pl.ds(base, bpw)], idx_v)
        rows = (rows0, rows1)
        gsem = (g0, g1)
        osem = (o0, o1)

        def _gather_refs(g, b):
            return (
                table_hbm.at[idx_v.at[pl.ds(g * CHUNK, CHUNK)]],
                rows[b],
            )

        def _wback_refs(g, b):
            return (
                rows[b],
                out_hbm.at[pl.ds(base + g * CHUNK, CHUNK)],
            )

        def gather_start(g, b):
            src, dst = _gather_refs(g, b)
            pltpu.async_copy(src, dst, gsem[b])

        def gather_wait(g, b):
            src, dst = _gather_refs(g, b)
            pltpu.make_async_copy(src, dst, gsem[b]).wait()

        def wback_start(g, b):
            src, dst = _wback_refs(g, b)
            pltpu.async_copy(src, dst, osem[b])

        def wback_wait(g, b):
            src, dst = _wback_refs(g, b)
            pltpu.make_async_copy(src, dst, osem[b]).wait()

        def scale(b):
            def scale_body(i, _):
                for j in range(D_MODEL // 16):
                    sl = pl.ds(j * 16, 16)
                    rows[b][i, sl] = rows[b][i, sl] * SQRT_D
                return _

            lax.fori_loop(0, CHUNK, scale_body, None)

        gather_start(0, 0)

        def pair_body(p, _):
            for b in range(2):
                g = 2 * p + b
                if b == 0:

                    @pl.when(p > 0)
                    def _():
                        wback_wait(g - 1, 1)

                    gather_start(g + 1, 1)
                else:

                    @pl.when(p < npair - 1)
                    def _():
                        wback_wait(g - 1, 0)
                        gather_start(g + 1, 0)

                gather_wait(g, b)
                scale(b)
                wback_start(g, b)
            return _

        lax.fori_loop(0, npair, pair_body, None)
        wback_wait(nchunk - 2, 0)
        wback_wait(nchunk - 1, 1)

    return k


def kernel(x, table):
    B = x.shape[0] * x.shape[1]
    idx = x.reshape(B).astype(jnp.int32)
    out = _make_kernel(B)(idx, table)
    return out.reshape(x.shape[0], x.shape[1], D_MODEL)
